# baseline (speedup 1.0000x reference)
import functools

import jax
import jax.numpy as jnp
from jax.experimental import pallas as pl
from jax.experimental.pallas import tpu as pltpu

_QB = 2000
_KB = 2048
_NK = 5
_NG = _KB // 128
_BIG = 1e30


def _nn_body(d1_ref, d2tm_ref, dist_ref, idx_ref, *, n_keys):
    d1 = d1_ref[...]

    best_val = jnp.full((_QB, 1), jnp.inf, jnp.float32)
    best_idx = jnp.zeros((_QB, 1), jnp.float32)
    pos = jax.lax.broadcasted_iota(
        jnp.int32, (1, 128), 1).astype(jnp.float32)

    for j in range(_NK):
        d2tm = d2tm_ref[:, j * _KB:(j + 1) * _KB]
        d2sq = 0.25 * jnp.sum(d2tm * d2tm, axis=0)
        kvec = jax.lax.broadcasted_iota(jnp.int32, (1, _KB), 1) + j * _KB
        d2sq = d2sq + jnp.where(kvec[0] < n_keys, 0.0, _BIG)
        dot = jnp.dot(d1, d2tm, preferred_element_type=jnp.float32)
        scores = dot + d2sq[None, :]

        colmin = scores[:, 0:128]
        for g in range(1, _NG):
            colmin = jnp.minimum(colmin, scores[:, g * 128:(g + 1) * 128])
        cmin = jnp.min(colmin, axis=1, keepdims=True)

        pstar = jnp.min(jnp.where(colmin == cmin, pos, jnp.float32(1e9)),
                        axis=1, keepdims=True)
        onehot = jnp.where(pos == pstar, 1.0, 0.0)

        gbest = jnp.full((_QB, 1), 0.0, jnp.float32)
        for g in range(_NG - 1, -1, -1):
            vg = jnp.sum(scores[:, g * 128:(g + 1) * 128] * onehot,
                         axis=1, keepdims=True)
            gbest = jnp.where(vg == cmin, jnp.float32(g), gbest)

        carg = gbest * 128.0 + pstar + jnp.float32(j * _KB)

        better = cmin < best_val
        best_idx = jnp.where(better, carg, best_idx)
        best_val = jnp.where(better, cmin, best_val)

    q_sq = jnp.sum(d1 * d1, axis=1, keepdims=True)
    dist_ref[...] = jnp.sqrt(jnp.maximum(best_val + q_sq, 0.0))
    row = (jax.lax.broadcasted_iota(jnp.int32, (_QB, 1), 0)
           + pl.program_id(0) * _QB)
    idx_ref[...] = jnp.concatenate(
        [row, best_idx.astype(jnp.int32)], axis=1)


def kernel(desc1, desc2):
    q, d = desc1.shape
    n_keys = desc2.shape[0]
    q_pad = ((q + _QB - 1) // _QB) * _QB
    k_pad = _NK * _KB
    d1p = jnp.pad(desc1, ((0, q_pad - q), (0, 0)))
    d2tm = (-2.0 * jnp.pad(desc2, ((0, k_pad - n_keys), (0, 0)))).T

    dists, idxs = pl.pallas_call(
        functools.partial(_nn_body, n_keys=n_keys),
        grid=(q_pad // _QB,),
        in_specs=[
            pl.BlockSpec((_QB, d), lambda i: (i, 0)),
            pl.BlockSpec((d, k_pad), lambda i: (0, 0)),
        ],
        out_specs=[
            pl.BlockSpec((_QB, 1), lambda i: (i, 0)),
            pl.BlockSpec((_QB, 2), lambda i: (i, 0)),
        ],
        out_shape=[
            jax.ShapeDtypeStruct((q_pad, 1), jnp.float32),
            jax.ShapeDtypeStruct((q_pad, 2), jnp.int32),
        ],
    )(d1p, d2tm)

    return dists[:q], idxs[:q]


# speedup vs baseline: 1.3664x; 1.3664x over previous
import functools

import jax
import jax.numpy as jnp
from jax.experimental import pallas as pl
from jax.experimental.pallas import tpu as pltpu

_QB = 2000
_KB = 2048
_NK = 5


def _nn_body(d1_ref, d2tm_ref, dist_ref, idx_ref, *, n_keys):
    d1 = d1_ref[...]

    best_val = jnp.full((_QB, 1), jnp.inf, jnp.float32)
    best_idx = jnp.zeros((_QB, 1), jnp.float32)
    lane = jax.lax.broadcasted_iota(
        jnp.int32, (1, _KB), 1).astype(jnp.float32)

    for j in range(_NK):
        d2tm = d2tm_ref[:, j * _KB:(j + 1) * _KB]
        d2sq = 0.25 * jnp.sum(d2tm * d2tm, axis=0)
        kvec = jax.lax.broadcasted_iota(jnp.int32, (1, _KB), 1) + j * _KB
        d2sq = d2sq + jnp.where(kvec[0] < n_keys, 0.0, jnp.inf)
        dot = jnp.dot(d1, d2tm, preferred_element_type=jnp.float32)
        scores = dot + d2sq[None, :]

        cmin = jnp.min(scores, axis=1, keepdims=True)
        carg = jnp.min(jnp.where(scores == cmin, lane, jnp.float32(2**30)),
                       axis=1, keepdims=True) + jnp.float32(j * _KB)

        better = cmin < best_val
        best_idx = jnp.where(better, carg, best_idx)
        best_val = jnp.where(better, cmin, best_val)

    q_sq = jnp.sum(d1 * d1, axis=1, keepdims=True)
    dist_ref[...] = jnp.sqrt(jnp.maximum(best_val + q_sq, 0.0))
    row = (jax.lax.broadcasted_iota(jnp.int32, (_QB, 1), 0)
           + pl.program_id(0) * _QB)
    idx_ref[...] = jnp.concatenate(
        [row, best_idx.astype(jnp.int32)], axis=1)


def kernel(desc1, desc2):
    q, d = desc1.shape
    n_keys = desc2.shape[0]
    q_pad = ((q + _QB - 1) // _QB) * _QB
    k_pad = _NK * _KB
    d1p = jnp.pad(desc1, ((0, q_pad - q), (0, 0)))
    d2tm = (-2.0 * jnp.pad(desc2, ((0, k_pad - n_keys), (0, 0)))).T

    dists, idxs = pl.pallas_call(
        functools.partial(_nn_body, n_keys=n_keys),
        grid=(q_pad // _QB,),
        in_specs=[
            pl.BlockSpec((_QB, d), lambda i: (i, 0)),
            pl.BlockSpec((d, k_pad), lambda i: (0, 0)),
        ],
        out_specs=[
            pl.BlockSpec((_QB, 1), lambda i: (i, 0)),
            pl.BlockSpec((_QB, 2), lambda i: (i, 0)),
        ],
        out_shape=[
            jax.ShapeDtypeStruct((q_pad, 1), jnp.float32),
            jax.ShapeDtypeStruct((q_pad, 2), jnp.int32),
        ],
    )(d1p, d2tm)

    return dists[:q], idxs[:q]
